# Initial kernel scaffold; baseline (speedup 1.0000x reference)
import jax
import jax.numpy as jnp
from jax.experimental import pallas as pl


def kernel(x, edge_index, batch, Wrel1, brel1, Wroot1, g1, b1, p1w, Wrel2, brel2, Wroot2, g2, b2, p2w, G1, B1, G2, B2, cw1, cb1, bg1, bb1, cw2, cb2, bg2, bb2, cw3, cb3, cw4, cb4, bg3, bb3, cw5, cb5, bg4, bb4, cw6, cb6, fcW, fcb, fc1W, fc1b):
    raise NotImplementedError("write your pallas kernel here")


# speedup vs baseline: 32.3729x; 32.3729x over previous
import functools

import jax
import jax.numpy as jnp
from jax.experimental import pallas as pl

G = 64
NPG = 200
NPP = 256
EPG = 3200
D = 128
H = 1024
K1, K1P = 100, 128
K2 = 50
N_ALL1 = float(G * NPG)
N_ALL2 = float(G * K1)

_f32 = jnp.float32


def _iota(shape, dim, dtype=jnp.int32):
    return jax.lax.broadcasted_iota(dtype, shape, dim)


def _dot(a, b):
    return jnp.dot(a, b, preferred_element_type=_f32,
                   precision=jax.lax.Precision.HIGHEST)


def _dotd(a, b):
    return jnp.dot(a.astype(jnp.bfloat16), b.astype(jnp.bfloat16),
                   preferred_element_type=_f32)


def _dgen(a, b, dims):
    return jax.lax.dot_general(a, b, (dims, ((), ())),
                               preferred_element_type=_f32,
                               precision=jax.lax.Precision.HIGHEST)


def _dgend(a, b, dims):
    return jax.lax.dot_general(a.astype(jnp.bfloat16), b.astype(jnp.bfloat16),
                               (dims, ((), ())),
                               preferred_element_type=_f32)


def _adj_kernel(src_ref, dst_ref, a_ref):
    sl = src_ref[0] % NPG
    dl = dst_ref[0] % NPG
    io = _iota((NPP, EPG), 0)
    od = (io == dl).astype(jnp.bfloat16)
    os_ = (io == sl).astype(jnp.bfloat16)
    a_ref[0] = jax.lax.dot_general(od, os_, ((((1,), (1,))), ((), ())),
                                   preferred_element_type=_f32)


def _gconv_kernel(n_valid, a_ref, x_ref, wr_ref, wo_ref, b_ref,
                  h_ref, s1_ref, s2_ref):
    a = a_ref[0]
    xg = x_ref[0]
    agg = _dot(a, xg)
    h = _dotd(agg, wr_ref[...]) + _dotd(xg, wo_ref[...]) + b_ref[...]
    rm = (_iota((h.shape[0], 1), 0) < n_valid).astype(_f32)
    h = h * rm
    h_ref[0] = h

    @pl.when(pl.program_id(0) == 0)
    def _():
        s1_ref[...] = jnp.zeros_like(s1_ref)
        s2_ref[...] = jnp.zeros_like(s2_ref)

    s1_ref[...] += jnp.sum(h, axis=0, keepdims=True)
    s2_ref[...] += jnp.sum(h * h, axis=0, keepdims=True)


def _bn_relu_rows(h, s1, s2, n_all, g, b, n_valid):
    mean = s1 / n_all
    var = s2 / n_all - mean * mean
    inv = jax.lax.rsqrt(var + 1e-5)
    hn = jnp.maximum((h - mean) * inv * g + b, 0.0)
    rm = (_iota((h.shape[0], 1), 0) < n_valid).astype(_f32)
    return hn * rm, rm


def _rank_row(scolm, n):
    eye = (_iota((n, n), 0) == _iota((n, n), 1)).astype(_f32)
    srowm = _dgen(scolm, eye, ((0,), (0,)))
    gt = (scolm > srowm).astype(_f32)
    eq = (scolm == srowm).astype(_f32)
    lt = (_iota((n, n), 0) < _iota((n, n), 1)).astype(_f32)
    return jnp.sum(gt + eq * lt, axis=0, keepdims=True)


def _pool1_kernel(h_ref, s1_ref, s2_ref, g_ref, b_ref, pw_ref, a_ref,
                  hp_ref, a2_ref, x1_ref):
    hn, rm = _bn_relu_rows(h_ref[0], s1_ref[...], s2_ref[...], N_ALL1,
                           g_ref[...], b_ref[...], NPG)
    pw = pw_ref[...]
    nrm = jnp.sqrt(jnp.sum(pw * pw))
    scol = jnp.tanh(_dotd(hn, pw) / nrm)
    scolm = jnp.where(rm > 0, scol, -1e30)
    rankrow = _rank_row(scolm, NPP)
    riota = _iota((K1P, NPP), 0).astype(_f32)
    p = jnp.where((riota == rankrow) & (rankrow < float(K1)), 1.0, 0.0)
    hs = hn * scol
    hp = _dot(p, hs)
    hp_ref[0] = hp
    x1_ref[0] = jnp.sum(hp, axis=0, keepdims=True) / float(K1)
    bmat = _dgen(a_ref[0], p, ((1,), (1,)))
    a2_ref[0] = _dot(p, bmat)


def _pool2_kernel(h_ref, s1_ref, s2_ref, g_ref, b_ref, pw_ref, x2_ref):
    hn, rm = _bn_relu_rows(h_ref[0], s1_ref[...], s2_ref[...], N_ALL2,
                           g_ref[...], b_ref[...], K1)
    pw = pw_ref[...]
    nrm = jnp.sqrt(jnp.sum(pw * pw))
    scol = jnp.tanh(_dotd(hn, pw) / nrm)
    scolm = jnp.where(rm > 0, scol, -1e30)
    rankrow = _rank_row(scolm, K1P)
    kept = jnp.where(rankrow < float(K2), 1.0, 0.0)
    hs = hn * scol
    x2_ref[0] = _dot(kept, hs) / float(K2)


def _pairmax(z, n_in):
    i0 = _iota((n_in, n_in // 2), 0)
    i1 = _iota((n_in, n_in // 2), 1)
    se = jnp.where(i0 == 2 * i1, 1.0, 0.0)
    so = jnp.where(i0 == 2 * i1 + 1, 1.0, 0.0)
    return jnp.maximum(_dot(z, se), _dot(z, so))


def _bn_batch_relu(m, g, b):
    mean = jnp.sum(m, axis=0, keepdims=True) / float(G)
    var = jnp.sum(m * m, axis=0, keepdims=True) / float(G) - mean * mean
    return jnp.maximum((m - mean) * jax.lax.rsqrt(var + 1e-5) * g + b, 0.0)


def _zprep_kernel(x1_ref, x2_ref, g1_ref, b1_ref, g2_ref, b2_ref, z_ref):
    z = x1_ref[...] + x2_ref[...]
    m = _pairmax(z, H)
    m = _bn_batch_relu(m, g1_ref[...], b1_ref[...])
    m = _pairmax(m, 512)
    z_ref[...] = _bn_batch_relu(m, g2_ref[...], b2_ref[...])


_NL = G * 256


def _shift(x, d):
    if d == 0:
        return x
    pad = jnp.zeros((x.shape[0], d), _f32)
    return jnp.concatenate([x[:, d:], pad], axis=1)


def _bn2d_masked(a, mask, count, g, b):
    am = a * mask
    mean = jnp.sum(am, axis=1, keepdims=True) / count
    var = jnp.sum(am * am, axis=1, keepdims=True) / count - mean * mean
    return (a - mean) * jax.lax.rsqrt(var + 1e-5) * g + b


def _head_kernel(z_ref, cw1_ref, cb1_ref, bg1_ref, bb1_ref, w2_ref, cb2_ref,
                 bg2_ref, bb2_ref, cw3_ref, cb3_ref, cw4_ref, cb4_ref,
                 bg3_ref, bb3_ref, w5_ref, cb5_ref, bg4_ref, bb4_ref,
                 cw6_ref, cb6_ref, out_ref):
    zrow = z_ref[...]
    q = _iota((1, _NL), 1) % 256
    hh = q // 16
    ww = q % 16
    m_a = jnp.where((hh % 2 == 0) & (ww % 2 == 0) & (hh <= 12) & (ww <= 12),
                    1.0, 0.0)
    m_b = jnp.where((hh % 4 == 0) & (ww % 4 == 0) & (hh <= 8) & (ww <= 8),
                    1.0, 0.0)

    a1 = jnp.maximum(_dot(cw1_ref[...], zrow) + cb1_ref[...], 0.0)
    ones = jnp.ones((1, _NL), _f32)
    a1 = _bn2d_masked(a1, ones, float(_NL), bg1_ref[...], bb1_ref[...])
    acc = jnp.zeros((64, _NL), _f32)
    for di in range(3):
        for dj in range(3):
            t = di * 3 + dj
            wt = w2_ref[t * 64:(t + 1) * 64, :]
            acc += _dotd(wt, _shift(a1, di * 16 + dj))
    a2 = jnp.maximum(acc + cb2_ref[...], 0.0)
    a2 = _bn2d_masked(a2, m_a, float(G * 49), bg2_ref[...], bb2_ref[...])
    a3 = jnp.maximum(_dotd(cw3_ref[...], a2) + cb3_ref[...], 0.0)
    mp = zrow
    first = True
    for di in range(3):
        for dj in range(3):
            s = _shift(zrow, di * 16 + dj)
            mp = s if first else jnp.maximum(mp, s)
            first = False
    z2 = (a3 + mp) * m_a

    a4 = jnp.maximum(_dot(cw4_ref[...], z2) + cb4_ref[...], 0.0)
    a4 = _bn2d_masked(a4, m_a, float(G * 49), bg3_ref[...], bb3_ref[...])
    acc5 = jnp.zeros((64, _NL), _f32)
    for di in range(3):
        for dj in range(3):
            t = di * 3 + dj
            wt = w5_ref[t * 64:(t + 1) * 64, :]
            acc5 += _dotd(wt, _shift(a4, 32 * di + 2 * dj))
    a5 = jnp.maximum(acc5 + cb5_ref[...], 0.0)
    a5 = _bn2d_masked(a5, m_b, float(G * 9), bg4_ref[...], bb4_ref[...])
    a6 = jnp.maximum(_dotd(cw6_ref[...], a5) + cb6_ref[...], 0.0)
    mp2 = z2
    first = True
    for di in range(3):
        for dj in range(3):
            s = _shift(z2, 32 * di + 2 * dj)
            mp2 = s if first else jnp.maximum(mp2, s)
            first = False
    out_ref[...] = (a6 + mp2) * m_b


def _fc_kernel(z_ref, fw_ref, fb_ref, f1w_ref, f1b_ref, out_ref):
    zm = z_ref[...]
    p_i = _iota((256, 16), 0)
    f_i = _iota((256, 16), 1)
    tgt = 64 * (f_i // 3) + 4 * (f_i % 3)
    sel = jnp.where((p_i == tgt) & (f_i < 9), 1.0, 0.0)
    z9 = _dot(zm, sel)
    hfc = jnp.maximum(_dotd(z9, fw_ref[...]) + fb_ref[...], 0.0)
    out_ref[...] = _dotd(hfc, f1w_ref[...]) + f1b_ref[...]


def kernel(x, edge_index, batch, Wrel1, brel1, Wroot1, g1, b1, p1w, Wrel2,
           brel2, Wroot2, g2, b2, p2w, G1, B1, G2, B2, cw1, cb1, bg1, bb1,
           cw2, cb2, bg2, bb2, cw3, cb3, cw4, cb4, bg3, bb3, cw5, cb5, bg4,
           bb4, cw6, cb6, fcW, fcb, fc1W, fc1b):
    f32 = _f32
    srcr = edge_index[0].reshape(G, 1, EPG)
    dstr = edge_index[1].reshape(G, 1, EPG)
    xp = jnp.pad(x.reshape(G, NPG, D), ((0, 0), (0, NPP - NPG), (0, 0)))

    adj = pl.pallas_call(
        _adj_kernel,
        grid=(G,),
        in_specs=[pl.BlockSpec((1, 1, EPG), lambda g: (g, 0, 0)),
                  pl.BlockSpec((1, 1, EPG), lambda g: (g, 0, 0))],
        out_specs=pl.BlockSpec((1, NPP, NPP), lambda g: (g, 0, 0)),
        out_shape=jax.ShapeDtypeStruct((G, NPP, NPP), f32),
    )(srcr, dstr)

    def gconv(a, h_in, n, wr, wo, b, n_valid):
        return pl.pallas_call(
            functools.partial(_gconv_kernel, n_valid),
            grid=(G,),
            in_specs=[
                pl.BlockSpec((1, n, n), lambda g: (g, 0, 0)),
                pl.BlockSpec((1, n, h_in.shape[-1]), lambda g: (g, 0, 0)),
                pl.BlockSpec(wr.shape, lambda g: (0, 0)),
                pl.BlockSpec(wo.shape, lambda g: (0, 0)),
                pl.BlockSpec((1, H), lambda g: (0, 0)),
            ],
            out_specs=[
                pl.BlockSpec((1, n, H), lambda g: (g, 0, 0)),
                pl.BlockSpec((1, H), lambda g: (0, 0)),
                pl.BlockSpec((1, H), lambda g: (0, 0)),
            ],
            out_shape=[
                jax.ShapeDtypeStruct((G, n, H), f32),
                jax.ShapeDtypeStruct((1, H), f32),
                jax.ShapeDtypeStruct((1, H), f32),
            ],
        )(a, h_in, wr, wo, b.reshape(1, H))

    h1, s1a, s2a = gconv(adj, xp, NPP, Wrel1, Wroot1, brel1, NPG)

    hp, adj2, x1 = pl.pallas_call(
        _pool1_kernel,
        grid=(G,),
        in_specs=[
            pl.BlockSpec((1, NPP, H), lambda g: (g, 0, 0)),
            pl.BlockSpec((1, H), lambda g: (0, 0)),
            pl.BlockSpec((1, H), lambda g: (0, 0)),
            pl.BlockSpec((1, H), lambda g: (0, 0)),
            pl.BlockSpec((1, H), lambda g: (0, 0)),
            pl.BlockSpec((H, 1), lambda g: (0, 0)),
            pl.BlockSpec((1, NPP, NPP), lambda g: (g, 0, 0)),
        ],
        out_specs=[
            pl.BlockSpec((1, K1P, H), lambda g: (g, 0, 0)),
            pl.BlockSpec((1, K1P, K1P), lambda g: (g, 0, 0)),
            pl.BlockSpec((1, 1, H), lambda g: (g, 0, 0)),
        ],
        out_shape=[
            jax.ShapeDtypeStruct((G, K1P, H), f32),
            jax.ShapeDtypeStruct((G, K1P, K1P), f32),
            jax.ShapeDtypeStruct((G, 1, H), f32),
        ],
    )(h1, s1a, s2a, g1.reshape(1, H), b1.reshape(1, H), p1w.reshape(H, 1), adj)

    h2, s1b, s2b = gconv(adj2, hp, K1P, Wrel2, Wroot2, brel2, K1)

    x2 = pl.pallas_call(
        _pool2_kernel,
        grid=(G,),
        in_specs=[
            pl.BlockSpec((1, K1P, H), lambda g: (g, 0, 0)),
            pl.BlockSpec((1, H), lambda g: (0, 0)),
            pl.BlockSpec((1, H), lambda g: (0, 0)),
            pl.BlockSpec((1, H), lambda g: (0, 0)),
            pl.BlockSpec((1, H), lambda g: (0, 0)),
            pl.BlockSpec((H, 1), lambda g: (0, 0)),
        ],
        out_specs=pl.BlockSpec((1, 1, H), lambda g: (g, 0, 0)),
        out_shape=jax.ShapeDtypeStruct((G, 1, H), f32),
    )(h2, s1b, s2b, g2.reshape(1, H), b2.reshape(1, H), p2w.reshape(H, 1))

    z256 = pl.pallas_call(
        _zprep_kernel,
        in_specs=[pl.BlockSpec(s, lambda: tuple([0] * len(s)))
                  for s in [(G, H), (G, H), (1, 512), (1, 512),
                            (1, 256), (1, 256)]],
        out_specs=pl.BlockSpec((G, 256), lambda: (0, 0)),
        out_shape=jax.ShapeDtypeStruct((G, 256), f32),
    )(x1.reshape(G, H), x2.reshape(G, H), G1.reshape(1, 512), B1.reshape(1, 512),
      G2.reshape(1, 256), B2.reshape(1, 256))

    zrow = z256.reshape(1, _NL)
    w2taps = cw2.transpose(2, 3, 0, 1).reshape(9 * 64, 128)
    w5taps = cw5.transpose(2, 3, 0, 1).reshape(9 * 64, 128)
    head_ins = [
        zrow, cw1.reshape(128, 1), cb1.reshape(128, 1),
        bg1.reshape(128, 1), bb1.reshape(128, 1),
        w2taps, cb2.reshape(64, 1), bg2.reshape(64, 1), bb2.reshape(64, 1),
        cw3.reshape(1, 64), cb3.reshape(1, 1),
        cw4.reshape(128, 1), cb4.reshape(128, 1),
        bg3.reshape(128, 1), bb3.reshape(128, 1),
        w5taps, cb5.reshape(64, 1), bg4.reshape(64, 1), bb4.reshape(64, 1),
        cw6.reshape(1, 64), cb6.reshape(1, 1),
    ]
    z2row = pl.pallas_call(
        _head_kernel,
        in_specs=[pl.BlockSpec(a.shape, lambda: tuple([0] * a.ndim))
                  for a in head_ins],
        out_specs=pl.BlockSpec((1, _NL), lambda: (0, 0)),
        out_shape=jax.ShapeDtypeStruct((1, _NL), f32),
    )(*head_ins)

    fcWp = jnp.pad(fcW, ((0, 7), (0, 0)))
    f1wp = jnp.pad(fc1W, ((0, 0), (0, 118)))
    f1bp = jnp.pad(fc1b.reshape(1, 10), ((0, 0), (0, 118)))
    outp = pl.pallas_call(
        _fc_kernel,
        in_specs=[
            pl.BlockSpec((G, 256), lambda: (0, 0)),
            pl.BlockSpec((16, H), lambda: (0, 0)),
            pl.BlockSpec((1, H), lambda: (0, 0)),
            pl.BlockSpec((H, 128), lambda: (0, 0)),
            pl.BlockSpec((1, 128), lambda: (0, 0)),
        ],
        out_specs=pl.BlockSpec((G, 128), lambda: (0, 0)),
        out_shape=jax.ShapeDtypeStruct((G, 128), f32),
    )(z2row.reshape(G, 256), fcWp, fcb.reshape(1, H), f1wp, f1bp)
    return outp[:, :10]


# speedup vs baseline: 32.4767x; 1.0032x over previous
import functools

import jax
import jax.numpy as jnp
from jax import lax
from jax.experimental import pallas as pl
from jax.experimental.pallas import tpu as pltpu
from jax.experimental.pallas import tpu_sc as plsc

G = 64
NPG = 200
NPP = 256
EPG = 3200
D = 128
H = 1024
K1, K1P = 100, 128
K2 = 50
N_ALL1 = float(G * NPG)
N_ALL2 = float(G * K1)

_f32 = jnp.float32


def _iota(shape, dim, dtype=jnp.int32):
    return jax.lax.broadcasted_iota(dtype, shape, dim)


def _dot(a, b):
    return jnp.dot(a, b, preferred_element_type=_f32,
                   precision=jax.lax.Precision.HIGHEST)


def _dotd(a, b):
    return jnp.dot(a.astype(jnp.bfloat16), b.astype(jnp.bfloat16),
                   preferred_element_type=_f32)


def _dgen(a, b, dims):
    return jax.lax.dot_general(a, b, (dims, ((), ())),
                               preferred_element_type=_f32,
                               precision=jax.lax.Precision.HIGHEST)


def _dgend(a, b, dims):
    return jax.lax.dot_general(a.astype(jnp.bfloat16), b.astype(jnp.bfloat16),
                               (dims, ((), ())),
                               preferred_element_type=_f32)


_FLAT = NPP * NPP
_adj_mesh = plsc.VectorSubcoreMesh(core_axis_name="c", subcore_axis_name="s")


@functools.partial(
    pl.kernel, mesh=_adj_mesh,
    compiler_params=pltpu.CompilerParams(needs_layout_passes=False),
    out_type=jax.ShapeDtypeStruct((G, _FLAT), jnp.float32),
    scratch_types=[
        pltpu.VMEM((EPG,), jnp.int32),
        pltpu.VMEM((EPG,), jnp.int32),
        pltpu.VMEM((_FLAT,), jnp.float32),
    ],
)
def _adj_sc(src_hbm, dst_hbm, out_hbm, src_v, dst_v, acc_v):
    wid = lax.axis_index("s") * 2 + lax.axis_index("c")
    zeros16 = jnp.zeros((16,), jnp.float32)
    for t in range(2):
        g = wid * 2 + t
        pltpu.sync_copy(src_hbm.at[g], src_v)
        pltpu.sync_copy(dst_hbm.at[g], dst_v)

        def zero_body(i, _):
            acc_v[pl.ds(i * 16, 16)] = zeros16
            return 0

        lax.fori_loop(0, _FLAT // 16, zero_body, 0, unroll=8)
        base = g * NPG

        def edge_body(c, _):
            sl = src_v[pl.ds(c * 16, 16)] - base
            dl = dst_v[pl.ds(c * 16, 16)] - base
            flat = dl * NPP + sl
            cnt, last = plsc.scan_count(flat)
            plsc.addupdate_scatter(
                acc_v, [flat], cnt.astype(jnp.float32), mask=last)
            return 0

        lax.fori_loop(0, EPG // 16, edge_body, 0, unroll=4)
        pltpu.sync_copy(acc_v, out_hbm.at[g])


def _gconv_kernel(n_valid, a_ref, x_ref, wr_ref, wo_ref, b_ref,
                  h_ref, s1_ref, s2_ref):
    a = a_ref[0]
    xg = x_ref[0]
    agg = _dot(a, xg)
    h = _dotd(agg, wr_ref[...]) + _dotd(xg, wo_ref[...]) + b_ref[...]
    rm = (_iota((h.shape[0], 1), 0) < n_valid).astype(_f32)
    h = h * rm
    h_ref[0] = h

    @pl.when(pl.program_id(0) == 0)
    def _():
        s1_ref[...] = jnp.zeros_like(s1_ref)
        s2_ref[...] = jnp.zeros_like(s2_ref)

    s1_ref[...] += jnp.sum(h, axis=0, keepdims=True)
    s2_ref[...] += jnp.sum(h * h, axis=0, keepdims=True)


def _bn_relu_rows(h, s1, s2, n_all, g, b, n_valid):
    mean = s1 / n_all
    var = s2 / n_all - mean * mean
    inv = jax.lax.rsqrt(var + 1e-5)
    hn = jnp.maximum((h - mean) * inv * g + b, 0.0)
    rm = (_iota((h.shape[0], 1), 0) < n_valid).astype(_f32)
    return hn * rm, rm


def _rank_row(scolm, n):
    eye = (_iota((n, n), 0) == _iota((n, n), 1)).astype(_f32)
    srowm = _dgen(scolm, eye, ((0,), (0,)))
    gt = (scolm > srowm).astype(_f32)
    eq = (scolm == srowm).astype(_f32)
    lt = (_iota((n, n), 0) < _iota((n, n), 1)).astype(_f32)
    return jnp.sum(gt + eq * lt, axis=0, keepdims=True)


def _pool1_kernel(h_ref, s1_ref, s2_ref, g_ref, b_ref, pw_ref, a_ref,
                  hp_ref, a2_ref, x1_ref):
    hn, rm = _bn_relu_rows(h_ref[0], s1_ref[...], s2_ref[...], N_ALL1,
                           g_ref[...], b_ref[...], NPG)
    pw = pw_ref[...]
    nrm = jnp.sqrt(jnp.sum(pw * pw))
    scol = jnp.tanh(_dotd(hn, pw) / nrm)
    scolm = jnp.where(rm > 0, scol, -1e30)
    rankrow = _rank_row(scolm, NPP)
    riota = _iota((K1P, NPP), 0).astype(_f32)
    p = jnp.where((riota == rankrow) & (rankrow < float(K1)), 1.0, 0.0)
    hs = hn * scol
    hp = _dot(p, hs)
    hp_ref[0] = hp
    x1_ref[0] = jnp.sum(hp, axis=0, keepdims=True) / float(K1)
    bmat = _dgen(a_ref[0], p, ((1,), (1,)))
    a2_ref[0] = _dot(p, bmat)


def _pool2_kernel(h_ref, s1_ref, s2_ref, g_ref, b_ref, pw_ref, x2_ref):
    hn, rm = _bn_relu_rows(h_ref[0], s1_ref[...], s2_ref[...], N_ALL2,
                           g_ref[...], b_ref[...], K1)
    pw = pw_ref[...]
    nrm = jnp.sqrt(jnp.sum(pw * pw))
    scol = jnp.tanh(_dotd(hn, pw) / nrm)
    scolm = jnp.where(rm > 0, scol, -1e30)
    rankrow = _rank_row(scolm, K1P)
    kept = jnp.where(rankrow < float(K2), 1.0, 0.0)
    hs = hn * scol
    x2_ref[0] = _dot(kept, hs) / float(K2)


def _pairmax(z, n_in):
    i0 = _iota((n_in, n_in // 2), 0)
    i1 = _iota((n_in, n_in // 2), 1)
    se = jnp.where(i0 == 2 * i1, 1.0, 0.0)
    so = jnp.where(i0 == 2 * i1 + 1, 1.0, 0.0)
    return jnp.maximum(_dot(z, se), _dot(z, so))


def _bn_batch_relu(m, g, b):
    mean = jnp.sum(m, axis=0, keepdims=True) / float(G)
    var = jnp.sum(m * m, axis=0, keepdims=True) / float(G) - mean * mean
    return jnp.maximum((m - mean) * jax.lax.rsqrt(var + 1e-5) * g + b, 0.0)


def _zprep_kernel(x1_ref, x2_ref, g1_ref, b1_ref, g2_ref, b2_ref, z_ref):
    z = x1_ref[...] + x2_ref[...]
    m = _pairmax(z, H)
    m = _bn_batch_relu(m, g1_ref[...], b1_ref[...])
    m = _pairmax(m, 512)
    z_ref[...] = _bn_batch_relu(m, g2_ref[...], b2_ref[...])


_NL = G * 256


def _shift(x, d):
    if d == 0:
        return x
    pad = jnp.zeros((x.shape[0], d), _f32)
    return jnp.concatenate([x[:, d:], pad], axis=1)


def _bn2d_masked(a, mask, count, g, b):
    am = a * mask
    mean = jnp.sum(am, axis=1, keepdims=True) / count
    var = jnp.sum(am * am, axis=1, keepdims=True) / count - mean * mean
    return (a - mean) * jax.lax.rsqrt(var + 1e-5) * g + b


def _head_kernel(z_ref, cw1_ref, cb1_ref, bg1_ref, bb1_ref, w2_ref, cb2_ref,
                 bg2_ref, bb2_ref, cw3_ref, cb3_ref, cw4_ref, cb4_ref,
                 bg3_ref, bb3_ref, w5_ref, cb5_ref, bg4_ref, bb4_ref,
                 cw6_ref, cb6_ref, out_ref):
    zrow = z_ref[...]
    q = _iota((1, _NL), 1) % 256
    hh = q // 16
    ww = q % 16
    m_a = jnp.where((hh % 2 == 0) & (ww % 2 == 0) & (hh <= 12) & (ww <= 12),
                    1.0, 0.0)
    m_b = jnp.where((hh % 4 == 0) & (ww % 4 == 0) & (hh <= 8) & (ww <= 8),
                    1.0, 0.0)

    a1 = jnp.maximum(_dot(cw1_ref[...], zrow) + cb1_ref[...], 0.0)
    ones = jnp.ones((1, _NL), _f32)
    a1 = _bn2d_masked(a1, ones, float(_NL), bg1_ref[...], bb1_ref[...])
    acc = jnp.zeros((64, _NL), _f32)
    for di in range(3):
        for dj in range(3):
            t = di * 3 + dj
            wt = w2_ref[t * 64:(t + 1) * 64, :]
            acc += _dotd(wt, _shift(a1, di * 16 + dj))
    a2 = jnp.maximum(acc + cb2_ref[...], 0.0)
    a2 = _bn2d_masked(a2, m_a, float(G * 49), bg2_ref[...], bb2_ref[...])
    a3 = jnp.maximum(_dotd(cw3_ref[...], a2) + cb3_ref[...], 0.0)
    mp = zrow
    first = True
    for di in range(3):
        for dj in range(3):
            s = _shift(zrow, di * 16 + dj)
            mp = s if first else jnp.maximum(mp, s)
            first = False
    z2 = (a3 + mp) * m_a

    a4 = jnp.maximum(_dot(cw4_ref[...], z2) + cb4_ref[...], 0.0)
    a4 = _bn2d_masked(a4, m_a, float(G * 49), bg3_ref[...], bb3_ref[...])
    acc5 = jnp.zeros((64, _NL), _f32)
    for di in range(3):
        for dj in range(3):
            t = di * 3 + dj
            wt = w5_ref[t * 64:(t + 1) * 64, :]
            acc5 += _dotd(wt, _shift(a4, 32 * di + 2 * dj))
    a5 = jnp.maximum(acc5 + cb5_ref[...], 0.0)
    a5 = _bn2d_masked(a5, m_b, float(G * 9), bg4_ref[...], bb4_ref[...])
    a6 = jnp.maximum(_dotd(cw6_ref[...], a5) + cb6_ref[...], 0.0)
    mp2 = z2
    first = True
    for di in range(3):
        for dj in range(3):
            s = _shift(z2, 32 * di + 2 * dj)
            mp2 = s if first else jnp.maximum(mp2, s)
            first = False
    out_ref[...] = (a6 + mp2) * m_b


def _fc_kernel(z_ref, fw_ref, fb_ref, f1w_ref, f1b_ref, out_ref):
    zm = z_ref[...]
    p_i = _iota((256, 16), 0)
    f_i = _iota((256, 16), 1)
    tgt = 64 * (f_i // 3) + 4 * (f_i % 3)
    sel = jnp.where((p_i == tgt) & (f_i < 9), 1.0, 0.0)
    z9 = _dot(zm, sel)
    hfc = jnp.maximum(_dotd(z9, fw_ref[...]) + fb_ref[...], 0.0)
    out_ref[...] = _dotd(hfc, f1w_ref[...]) + f1b_ref[...]


def kernel(x, edge_index, batch, Wrel1, brel1, Wroot1, g1, b1, p1w, Wrel2,
           brel2, Wroot2, g2, b2, p2w, G1, B1, G2, B2, cw1, cb1, bg1, bb1,
           cw2, cb2, bg2, bb2, cw3, cb3, cw4, cb4, bg3, bb3, cw5, cb5, bg4,
           bb4, cw6, cb6, fcW, fcb, fc1W, fc1b):
    f32 = _f32
    xp = jnp.pad(x.reshape(G, NPG, D), ((0, 0), (0, NPP - NPG), (0, 0)))

    adj = _adj_sc(edge_index[0].reshape(G, EPG),
                  edge_index[1].reshape(G, EPG)).reshape(G, NPP, NPP)

    def gconv(a, h_in, n, wr, wo, b, n_valid):
        return pl.pallas_call(
            functools.partial(_gconv_kernel, n_valid),
            grid=(G,),
            in_specs=[
                pl.BlockSpec((1, n, n), lambda g: (g, 0, 0)),
                pl.BlockSpec((1, n, h_in.shape[-1]), lambda g: (g, 0, 0)),
                pl.BlockSpec(wr.shape, lambda g: (0, 0)),
                pl.BlockSpec(wo.shape, lambda g: (0, 0)),
                pl.BlockSpec((1, H), lambda g: (0, 0)),
            ],
            out_specs=[
                pl.BlockSpec((1, n, H), lambda g: (g, 0, 0)),
                pl.BlockSpec((1, H), lambda g: (0, 0)),
                pl.BlockSpec((1, H), lambda g: (0, 0)),
            ],
            out_shape=[
                jax.ShapeDtypeStruct((G, n, H), f32),
                jax.ShapeDtypeStruct((1, H), f32),
                jax.ShapeDtypeStruct((1, H), f32),
            ],
        )(a, h_in, wr, wo, b.reshape(1, H))

    h1, s1a, s2a = gconv(adj, xp, NPP, Wrel1, Wroot1, brel1, NPG)

    hp, adj2, x1 = pl.pallas_call(
        _pool1_kernel,
        grid=(G,),
        in_specs=[
            pl.BlockSpec((1, NPP, H), lambda g: (g, 0, 0)),
            pl.BlockSpec((1, H), lambda g: (0, 0)),
            pl.BlockSpec((1, H), lambda g: (0, 0)),
            pl.BlockSpec((1, H), lambda g: (0, 0)),
            pl.BlockSpec((1, H), lambda g: (0, 0)),
            pl.BlockSpec((H, 1), lambda g: (0, 0)),
            pl.BlockSpec((1, NPP, NPP), lambda g: (g, 0, 0)),
        ],
        out_specs=[
            pl.BlockSpec((1, K1P, H), lambda g: (g, 0, 0)),
            pl.BlockSpec((1, K1P, K1P), lambda g: (g, 0, 0)),
            pl.BlockSpec((1, 1, H), lambda g: (g, 0, 0)),
        ],
        out_shape=[
            jax.ShapeDtypeStruct((G, K1P, H), f32),
            jax.ShapeDtypeStruct((G, K1P, K1P), f32),
            jax.ShapeDtypeStruct((G, 1, H), f32),
        ],
    )(h1, s1a, s2a, g1.reshape(1, H), b1.reshape(1, H), p1w.reshape(H, 1), adj)

    h2, s1b, s2b = gconv(adj2, hp, K1P, Wrel2, Wroot2, brel2, K1)

    x2 = pl.pallas_call(
        _pool2_kernel,
        grid=(G,),
        in_specs=[
            pl.BlockSpec((1, K1P, H), lambda g: (g, 0, 0)),
            pl.BlockSpec((1, H), lambda g: (0, 0)),
            pl.BlockSpec((1, H), lambda g: (0, 0)),
            pl.BlockSpec((1, H), lambda g: (0, 0)),
            pl.BlockSpec((1, H), lambda g: (0, 0)),
            pl.BlockSpec((H, 1), lambda g: (0, 0)),
        ],
        out_specs=pl.BlockSpec((1, 1, H), lambda g: (g, 0, 0)),
        out_shape=jax.ShapeDtypeStruct((G, 1, H), f32),
    )(h2, s1b, s2b, g2.reshape(1, H), b2.reshape(1, H), p2w.reshape(H, 1))

    z256 = pl.pallas_call(
        _zprep_kernel,
        in_specs=[pl.BlockSpec(s, lambda: tuple([0] * len(s)))
                  for s in [(G, H), (G, H), (1, 512), (1, 512),
                            (1, 256), (1, 256)]],
        out_specs=pl.BlockSpec((G, 256), lambda: (0, 0)),
        out_shape=jax.ShapeDtypeStruct((G, 256), f32),
    )(x1.reshape(G, H), x2.reshape(G, H), G1.reshape(1, 512), B1.reshape(1, 512),
      G2.reshape(1, 256), B2.reshape(1, 256))

    zrow = z256.reshape(1, _NL)
    w2taps = cw2.transpose(2, 3, 0, 1).reshape(9 * 64, 128)
    w5taps = cw5.transpose(2, 3, 0, 1).reshape(9 * 64, 128)
    head_ins = [
        zrow, cw1.reshape(128, 1), cb1.reshape(128, 1),
        bg1.reshape(128, 1), bb1.reshape(128, 1),
        w2taps, cb2.reshape(64, 1), bg2.reshape(64, 1), bb2.reshape(64, 1),
        cw3.reshape(1, 64), cb3.reshape(1, 1),
        cw4.reshape(128, 1), cb4.reshape(128, 1),
        bg3.reshape(128, 1), bb3.reshape(128, 1),
        w5taps, cb5.reshape(64, 1), bg4.reshape(64, 1), bb4.reshape(64, 1),
        cw6.reshape(1, 64), cb6.reshape(1, 1),
    ]
    z2row = pl.pallas_call(
        _head_kernel,
        in_specs=[pl.BlockSpec(a.shape, lambda: tuple([0] * a.ndim))
                  for a in head_ins],
        out_specs=pl.BlockSpec((1, _NL), lambda: (0, 0)),
        out_shape=jax.ShapeDtypeStruct((1, _NL), f32),
    )(*head_ins)

    fcWp = jnp.pad(fcW, ((0, 7), (0, 0)))
    f1wp = jnp.pad(fc1W, ((0, 0), (0, 118)))
    f1bp = jnp.pad(fc1b.reshape(1, 10), ((0, 0), (0, 118)))
    outp = pl.pallas_call(
        _fc_kernel,
        in_specs=[
            pl.BlockSpec((G, 256), lambda: (0, 0)),
            pl.BlockSpec((16, H), lambda: (0, 0)),
            pl.BlockSpec((1, H), lambda: (0, 0)),
            pl.BlockSpec((H, 128), lambda: (0, 0)),
            pl.BlockSpec((1, 128), lambda: (0, 0)),
        ],
        out_specs=pl.BlockSpec((G, 128), lambda: (0, 0)),
        out_shape=jax.ShapeDtypeStruct((G, 128), f32),
    )(z2row.reshape(G, 256), fcWp, fcb.reshape(1, H), f1wp, f1bp)
    return outp[:, :10]


# speedup vs baseline: 34.7270x; 1.0693x over previous
import functools

import jax
import jax.numpy as jnp
from jax import lax
from jax.experimental import pallas as pl
from jax.experimental.pallas import tpu as pltpu
from jax.experimental.pallas import tpu_sc as plsc

G = 64
NPG = 200
NPP = 256
EPG = 3200
D = 128
H = 1024
K1, K1P = 100, 128
K2 = 50
N_ALL1 = float(G * NPG)
N_ALL2 = float(G * K1)

_f32 = jnp.float32


def _iota(shape, dim, dtype=jnp.int32):
    return jax.lax.broadcasted_iota(dtype, shape, dim)


def _dot(a, b):
    return jnp.dot(a, b, preferred_element_type=_f32,
                   precision=jax.lax.Precision.HIGHEST)


def _dotd(a, b):
    return jnp.dot(a.astype(jnp.bfloat16), b.astype(jnp.bfloat16),
                   preferred_element_type=_f32)


def _dgen(a, b, dims):
    return jax.lax.dot_general(a, b, (dims, ((), ())),
                               preferred_element_type=_f32,
                               precision=jax.lax.Precision.HIGHEST)


def _dgend(a, b, dims):
    return jax.lax.dot_general(a.astype(jnp.bfloat16), b.astype(jnp.bfloat16),
                               (dims, ((), ())),
                               preferred_element_type=_f32)


_FLAT = NPP * NPP
_adj_mesh = plsc.VectorSubcoreMesh(core_axis_name="c", subcore_axis_name="s")


@functools.partial(
    pl.kernel, mesh=_adj_mesh,
    compiler_params=pltpu.CompilerParams(needs_layout_passes=False),
    out_type=jax.ShapeDtypeStruct((G, _FLAT), jnp.float32),
    scratch_types=[
        pltpu.VMEM((EPG,), jnp.int32),
        pltpu.VMEM((EPG,), jnp.int32),
        pltpu.VMEM((_FLAT,), jnp.float32),
    ],
)
def _adj_sc(src_hbm, dst_hbm, out_hbm, src_v, dst_v, acc_v):
    wid = lax.axis_index("s") * 2 + lax.axis_index("c")
    zeros16 = jnp.zeros((16,), jnp.float32)
    for t in range(2):
        g = wid * 2 + t
        pltpu.sync_copy(src_hbm.at[g], src_v)
        pltpu.sync_copy(dst_hbm.at[g], dst_v)

        def zero_body(i, _):
            acc_v[pl.ds(i * 16, 16)] = zeros16
            return 0

        lax.fori_loop(0, _FLAT // 16, zero_body, 0, unroll=8)
        base = g * NPG

        def edge_body(c, _):
            sl = src_v[pl.ds(c * 16, 16)] - base
            dl = dst_v[pl.ds(c * 16, 16)] - base
            flat = dl * NPP + sl
            cnt, last = plsc.scan_count(flat)
            plsc.addupdate_scatter(
                acc_v, [flat], cnt.astype(jnp.float32), mask=last)
            return 0

        lax.fori_loop(0, EPG // 16, edge_body, 0, unroll=4)
        pltpu.sync_copy(acc_v, out_hbm.at[g])


def _gconv_kernel(n_valid, a_ref, x_ref, wr_ref, wo_ref, b_ref,
                  h_ref, s1_ref, s2_ref):
    a = a_ref[0]
    xg = x_ref[0]
    agg = _dot(a, xg)
    h = _dotd(agg, wr_ref[...]) + _dotd(xg, wo_ref[...]) + b_ref[...]
    rm = (_iota((h.shape[0], 1), 0) < n_valid).astype(_f32)
    h = h * rm
    h_ref[0] = h

    @pl.when(pl.program_id(0) == 0)
    def _():
        s1_ref[...] = jnp.zeros_like(s1_ref)
        s2_ref[...] = jnp.zeros_like(s2_ref)

    s1_ref[...] += jnp.sum(h, axis=0, keepdims=True)
    s2_ref[...] += jnp.sum(h * h, axis=0, keepdims=True)


def _bn_relu_rows(h, s1, s2, n_all, g, b, n_valid):
    mean = s1 / n_all
    var = s2 / n_all - mean * mean
    inv = jax.lax.rsqrt(var + 1e-5)
    hn = jnp.maximum((h - mean) * inv * g + b, 0.0)
    rm = (_iota((h.shape[0], 1), 0) < n_valid).astype(_f32)
    return hn * rm, rm


def _rank_row(scolm, n):
    eye = (_iota((n, n), 0) == _iota((n, n), 1)).astype(_f32)
    srowm = _dgen(scolm, eye, ((0,), (0,)))
    gt = (scolm > srowm).astype(_f32)
    eq = (scolm == srowm).astype(_f32)
    lt = (_iota((n, n), 0) < _iota((n, n), 1)).astype(_f32)
    return jnp.sum(gt + eq * lt, axis=0, keepdims=True)


def _pool1_kernel(h_ref, s1_ref, s2_ref, g_ref, b_ref, pw_ref, a_ref,
                  hp_ref, a2_ref, x1_ref):
    hn, rm = _bn_relu_rows(h_ref[0], s1_ref[...], s2_ref[...], N_ALL1,
                           g_ref[...], b_ref[...], NPG)
    pw = pw_ref[...]
    nrm = jnp.sqrt(jnp.sum(pw * pw))
    scol = jnp.tanh(_dotd(hn, pw) / nrm)
    scolm = jnp.where(rm > 0, scol, -1e30)
    rankrow = _rank_row(scolm, NPP)
    riota = _iota((K1P, NPP), 0).astype(_f32)
    p = jnp.where((riota == rankrow) & (rankrow < float(K1)), 1.0, 0.0)
    hs = hn * scol
    hp = _dot(p, hs)
    hp_ref[0] = hp
    x1_ref[0] = jnp.sum(hp, axis=0, keepdims=True) / float(K1)
    bmat = _dgend(a_ref[0], p, ((1,), (1,)))
    a2_ref[0] = _dotd(p, bmat)


def _pool2_kernel(h_ref, s1_ref, s2_ref, g_ref, b_ref, pw_ref, x2_ref):
    hn, rm = _bn_relu_rows(h_ref[0], s1_ref[...], s2_ref[...], N_ALL2,
                           g_ref[...], b_ref[...], K1)
    pw = pw_ref[...]
    nrm = jnp.sqrt(jnp.sum(pw * pw))
    scol = jnp.tanh(_dotd(hn, pw) / nrm)
    scolm = jnp.where(rm > 0, scol, -1e30)
    rankrow = _rank_row(scolm, K1P)
    kept = jnp.where(rankrow < float(K2), 1.0, 0.0)
    hs = hn * scol
    x2_ref[0] = _dot(kept, hs) / float(K2)


def _pairmax(z, n_in):
    i0 = _iota((n_in, n_in // 2), 0)
    i1 = _iota((n_in, n_in // 2), 1)
    se = jnp.where(i0 == 2 * i1, 1.0, 0.0)
    so = jnp.where(i0 == 2 * i1 + 1, 1.0, 0.0)
    return jnp.maximum(_dot(z, se), _dot(z, so))


def _bn_batch_relu(m, g, b):
    mean = jnp.sum(m, axis=0, keepdims=True) / float(G)
    var = jnp.sum(m * m, axis=0, keepdims=True) / float(G) - mean * mean
    return jnp.maximum((m - mean) * jax.lax.rsqrt(var + 1e-5) * g + b, 0.0)


def _zprep_kernel(x1_ref, x2_ref, g1_ref, b1_ref, g2_ref, b2_ref, z_ref):
    z = x1_ref[...] + x2_ref[...]
    m = _pairmax(z, H)
    m = _bn_batch_relu(m, g1_ref[...], b1_ref[...])
    m = _pairmax(m, 512)
    z_ref[...] = _bn_batch_relu(m, g2_ref[...], b2_ref[...])


_NL = G * 256


def _shift(x, d):
    if d == 0:
        return x
    pad = jnp.zeros((x.shape[0], d), _f32)
    return jnp.concatenate([x[:, d:], pad], axis=1)


def _bn2d_masked(a, mask, count, g, b):
    am = a * mask
    mean = jnp.sum(am, axis=1, keepdims=True) / count
    var = jnp.sum(am * am, axis=1, keepdims=True) / count - mean * mean
    return (a - mean) * jax.lax.rsqrt(var + 1e-5) * g + b


def _head_kernel(z_ref, cw1_ref, cb1_ref, bg1_ref, bb1_ref, w2_ref, cb2_ref,
                 bg2_ref, bb2_ref, cw3_ref, cb3_ref, cw4_ref, cb4_ref,
                 bg3_ref, bb3_ref, w5_ref, cb5_ref, bg4_ref, bb4_ref,
                 cw6_ref, cb6_ref, out_ref):
    zrow = z_ref[...]
    q = _iota((1, _NL), 1) % 256
    hh = q // 16
    ww = q % 16
    m_a = jnp.where((hh % 2 == 0) & (ww % 2 == 0) & (hh <= 12) & (ww <= 12),
                    1.0, 0.0)
    m_b = jnp.where((hh % 4 == 0) & (ww % 4 == 0) & (hh <= 8) & (ww <= 8),
                    1.0, 0.0)

    a1 = jnp.maximum(_dot(cw1_ref[...], zrow) + cb1_ref[...], 0.0)
    ones = jnp.ones((1, _NL), _f32)
    a1 = _bn2d_masked(a1, ones, float(_NL), bg1_ref[...], bb1_ref[...])
    acc = jnp.zeros((64, _NL), _f32)
    for di in range(3):
        for dj in range(3):
            t = di * 3 + dj
            wt = w2_ref[t * 64:(t + 1) * 64, :]
            acc += _dotd(wt, _shift(a1, di * 16 + dj))
    a2 = jnp.maximum(acc + cb2_ref[...], 0.0)
    a2 = _bn2d_masked(a2, m_a, float(G * 49), bg2_ref[...], bb2_ref[...])
    a3 = jnp.maximum(_dotd(cw3_ref[...], a2) + cb3_ref[...], 0.0)
    mp = zrow
    first = True
    for di in range(3):
        for dj in range(3):
            s = _shift(zrow, di * 16 + dj)
            mp = s if first else jnp.maximum(mp, s)
            first = False
    z2 = (a3 + mp) * m_a

    a4 = jnp.maximum(_dot(cw4_ref[...], z2) + cb4_ref[...], 0.0)
    a4 = _bn2d_masked(a4, m_a, float(G * 49), bg3_ref[...], bb3_ref[...])
    acc5 = jnp.zeros((64, _NL), _f32)
    for di in range(3):
        for dj in range(3):
            t = di * 3 + dj
            wt = w5_ref[t * 64:(t + 1) * 64, :]
            acc5 += _dotd(wt, _shift(a4, 32 * di + 2 * dj))
    a5 = jnp.maximum(acc5 + cb5_ref[...], 0.0)
    a5 = _bn2d_masked(a5, m_b, float(G * 9), bg4_ref[...], bb4_ref[...])
    a6 = jnp.maximum(_dotd(cw6_ref[...], a5) + cb6_ref[...], 0.0)
    mp2 = z2
    first = True
    for di in range(3):
        for dj in range(3):
            s = _shift(z2, 32 * di + 2 * dj)
            mp2 = s if first else jnp.maximum(mp2, s)
            first = False
    out_ref[...] = (a6 + mp2) * m_b


def _fc_kernel(z_ref, fw_ref, fb_ref, f1w_ref, f1b_ref, out_ref):
    zm = z_ref[...]
    p_i = _iota((256, 16), 0)
    f_i = _iota((256, 16), 1)
    tgt = 64 * (f_i // 3) + 4 * (f_i % 3)
    sel = jnp.where((p_i == tgt) & (f_i < 9), 1.0, 0.0)
    z9 = _dot(zm, sel)
    hfc = jnp.maximum(_dotd(z9, fw_ref[...]) + fb_ref[...], 0.0)
    out_ref[...] = _dotd(hfc, f1w_ref[...]) + f1b_ref[...]


def kernel(x, edge_index, batch, Wrel1, brel1, Wroot1, g1, b1, p1w, Wrel2,
           brel2, Wroot2, g2, b2, p2w, G1, B1, G2, B2, cw1, cb1, bg1, bb1,
           cw2, cb2, bg2, bb2, cw3, cb3, cw4, cb4, bg3, bb3, cw5, cb5, bg4,
           bb4, cw6, cb6, fcW, fcb, fc1W, fc1b):
    f32 = _f32
    xp = jnp.pad(x.reshape(G, NPG, D), ((0, 0), (0, NPP - NPG), (0, 0)))

    adj = _adj_sc(edge_index[0].reshape(G, EPG),
                  edge_index[1].reshape(G, EPG)).reshape(G, NPP, NPP)

    def gconv(a, h_in, n, wr, wo, b, n_valid):
        return pl.pallas_call(
            functools.partial(_gconv_kernel, n_valid),
            grid=(G,),
            in_specs=[
                pl.BlockSpec((1, n, n), lambda g: (g, 0, 0)),
                pl.BlockSpec((1, n, h_in.shape[-1]), lambda g: (g, 0, 0)),
                pl.BlockSpec(wr.shape, lambda g: (0, 0)),
                pl.BlockSpec(wo.shape, lambda g: (0, 0)),
                pl.BlockSpec((1, H), lambda g: (0, 0)),
            ],
            out_specs=[
                pl.BlockSpec((1, n, H), lambda g: (g, 0, 0)),
                pl.BlockSpec((1, H), lambda g: (0, 0)),
                pl.BlockSpec((1, H), lambda g: (0, 0)),
            ],
            out_shape=[
                jax.ShapeDtypeStruct((G, n, H), f32),
                jax.ShapeDtypeStruct((1, H), f32),
                jax.ShapeDtypeStruct((1, H), f32),
            ],
        )(a, h_in, wr, wo, b.reshape(1, H))

    h1, s1a, s2a = gconv(adj, xp, NPP, Wrel1, Wroot1, brel1, NPG)

    hp, adj2, x1 = pl.pallas_call(
        _pool1_kernel,
        grid=(G,),
        in_specs=[
            pl.BlockSpec((1, NPP, H), lambda g: (g, 0, 0)),
            pl.BlockSpec((1, H), lambda g: (0, 0)),
            pl.BlockSpec((1, H), lambda g: (0, 0)),
            pl.BlockSpec((1, H), lambda g: (0, 0)),
            pl.BlockSpec((1, H), lambda g: (0, 0)),
            pl.BlockSpec((H, 1), lambda g: (0, 0)),
            pl.BlockSpec((1, NPP, NPP), lambda g: (g, 0, 0)),
        ],
        out_specs=[
            pl.BlockSpec((1, K1P, H), lambda g: (g, 0, 0)),
            pl.BlockSpec((1, K1P, K1P), lambda g: (g, 0, 0)),
            pl.BlockSpec((1, 1, H), lambda g: (g, 0, 0)),
        ],
        out_shape=[
            jax.ShapeDtypeStruct((G, K1P, H), f32),
            jax.ShapeDtypeStruct((G, K1P, K1P), f32),
            jax.ShapeDtypeStruct((G, 1, H), f32),
        ],
    )(h1, s1a, s2a, g1.reshape(1, H), b1.reshape(1, H), p1w.reshape(H, 1), adj)

    h2, s1b, s2b = gconv(adj2, hp, K1P, Wrel2, Wroot2, brel2, K1)

    x2 = pl.pallas_call(
        _pool2_kernel,
        grid=(G,),
        in_specs=[
            pl.BlockSpec((1, K1P, H), lambda g: (g, 0, 0)),
            pl.BlockSpec((1, H), lambda g: (0, 0)),
            pl.BlockSpec((1, H), lambda g: (0, 0)),
            pl.BlockSpec((1, H), lambda g: (0, 0)),
            pl.BlockSpec((1, H), lambda g: (0, 0)),
            pl.BlockSpec((H, 1), lambda g: (0, 0)),
        ],
        out_specs=pl.BlockSpec((1, 1, H), lambda g: (g, 0, 0)),
        out_shape=jax.ShapeDtypeStruct((G, 1, H), f32),
    )(h2, s1b, s2b, g2.reshape(1, H), b2.reshape(1, H), p2w.reshape(H, 1))

    z256 = pl.pallas_call(
        _zprep_kernel,
        in_specs=[pl.BlockSpec(s, lambda: tuple([0] * len(s)))
                  for s in [(G, H), (G, H), (1, 512), (1, 512),
                            (1, 256), (1, 256)]],
        out_specs=pl.BlockSpec((G, 256), lambda: (0, 0)),
        out_shape=jax.ShapeDtypeStruct((G, 256), f32),
    )(x1.reshape(G, H), x2.reshape(G, H), G1.reshape(1, 512), B1.reshape(1, 512),
      G2.reshape(1, 256), B2.reshape(1, 256))

    zrow = z256.reshape(1, _NL)
    w2taps = cw2.transpose(2, 3, 0, 1).reshape(9 * 64, 128)
    w5taps = cw5.transpose(2, 3, 0, 1).reshape(9 * 64, 128)
    head_ins = [
        zrow, cw1.reshape(128, 1), cb1.reshape(128, 1),
        bg1.reshape(128, 1), bb1.reshape(128, 1),
        w2taps, cb2.reshape(64, 1), bg2.reshape(64, 1), bb2.reshape(64, 1),
        cw3.reshape(1, 64), cb3.reshape(1, 1),
        cw4.reshape(128, 1), cb4.reshape(128, 1),
        bg3.reshape(128, 1), bb3.reshape(128, 1),
        w5taps, cb5.reshape(64, 1), bg4.reshape(64, 1), bb4.reshape(64, 1),
        cw6.reshape(1, 64), cb6.reshape(1, 1),
    ]
    z2row = pl.pallas_call(
        _head_kernel,
        in_specs=[pl.BlockSpec(a.shape, lambda: tuple([0] * a.ndim))
                  for a in head_ins],
        out_specs=pl.BlockSpec((1, _NL), lambda: (0, 0)),
        out_shape=jax.ShapeDtypeStruct((1, _NL), f32),
    )(*head_ins)

    fcWp = jnp.pad(fcW, ((0, 7), (0, 0)))
    f1wp = jnp.pad(fc1W, ((0, 0), (0, 118)))
    f1bp = jnp.pad(fc1b.reshape(1, 10), ((0, 0), (0, 118)))
    outp = pl.pallas_call(
        _fc_kernel,
        in_specs=[
            pl.BlockSpec((G, 256), lambda: (0, 0)),
            pl.BlockSpec((16, H), lambda: (0, 0)),
            pl.BlockSpec((1, H), lambda: (0, 0)),
            pl.BlockSpec((H, 128), lambda: (0, 0)),
            pl.BlockSpec((1, 128), lambda: (0, 0)),
        ],
        out_specs=pl.BlockSpec((G, 128), lambda: (0, 0)),
        out_shape=jax.ShapeDtypeStruct((G, 128), f32),
    )(z2row.reshape(G, 256), fcWp, fcb.reshape(1, H), f1wp, f1bp)
    return outp[:, :10]


# speedup vs baseline: 37.6560x; 1.0843x over previous
import functools

import jax
import jax.numpy as jnp
from jax import lax
from jax.experimental import pallas as pl
from jax.experimental.pallas import tpu as pltpu
from jax.experimental.pallas import tpu_sc as plsc

G = 64
NPG = 200
NPP = 256
EPG = 3200
D = 128
H = 1024
K1, K1P = 100, 128
K2 = 50
N_ALL1 = float(G * NPG)
N_ALL2 = float(G * K1)

_f32 = jnp.float32


def _iota(shape, dim, dtype=jnp.int32):
    return jax.lax.broadcasted_iota(dtype, shape, dim)


def _dot(a, b):
    return jnp.dot(a, b, preferred_element_type=_f32,
                   precision=jax.lax.Precision.HIGHEST)


def _dotd(a, b):
    return jnp.dot(a.astype(jnp.bfloat16), b.astype(jnp.bfloat16),
                   preferred_element_type=_f32)


def _dgen(a, b, dims):
    return jax.lax.dot_general(a, b, (dims, ((), ())),
                               preferred_element_type=_f32,
                               precision=jax.lax.Precision.HIGHEST)


def _dgend(a, b, dims):
    return jax.lax.dot_general(a.astype(jnp.bfloat16), b.astype(jnp.bfloat16),
                               (dims, ((), ())),
                               preferred_element_type=_f32)


def _dote(a, b):
    bh = b.astype(jnp.bfloat16).astype(_f32)
    return _dotd(a, bh) + _dotd(a, b - bh)


_FLAT = NPP * NPP
_adj_mesh = plsc.VectorSubcoreMesh(core_axis_name="c", subcore_axis_name="s")


@functools.partial(
    pl.kernel, mesh=_adj_mesh,
    compiler_params=pltpu.CompilerParams(needs_layout_passes=False),
    out_type=jax.ShapeDtypeStruct((G, _FLAT), jnp.float32),
    scratch_types=[
        pltpu.VMEM((EPG,), jnp.int32),
        pltpu.VMEM((EPG,), jnp.int32),
        pltpu.VMEM((_FLAT,), jnp.float32),
    ],
)
def _adj_sc(src_hbm, dst_hbm, out_hbm, src_v, dst_v, acc_v):
    wid = lax.axis_index("s") * 2 + lax.axis_index("c")
    zeros16 = jnp.zeros((16,), jnp.float32)
    for t in range(2):
        g = wid * 2 + t
        pltpu.sync_copy(src_hbm.at[g], src_v)
        pltpu.sync_copy(dst_hbm.at[g], dst_v)

        def zero_body(i, _):
            acc_v[pl.ds(i * 16, 16)] = zeros16
            return 0

        lax.fori_loop(0, _FLAT // 16, zero_body, 0, unroll=8)
        base = g * NPG

        def edge_body(c, _):
            sl = src_v[pl.ds(c * 16, 16)] - base
            dl = dst_v[pl.ds(c * 16, 16)] - base
            flat = dl * NPP + sl
            cnt, last = plsc.scan_count(flat)
            plsc.addupdate_scatter(
                acc_v, [flat], cnt.astype(jnp.float32), mask=last)
            return 0

        lax.fori_loop(0, EPG // 16, edge_body, 0, unroll=4)
        pltpu.sync_copy(acc_v, out_hbm.at[g])


def _gconv_kernel(n_valid, a_ref, x_ref, wr_ref, wo_ref, b_ref,
                  h_ref, s1_ref, s2_ref):
    a = a_ref[0]
    xg = x_ref[0]
    agg = _dote(a, xg)
    h = _dotd(agg, wr_ref[...]) + _dotd(xg, wo_ref[...]) + b_ref[...]
    rm = (_iota((h.shape[0], 1), 0) < n_valid).astype(_f32)
    h = h * rm
    h_ref[0] = h

    @pl.when(pl.program_id(0) == 0)
    def _():
        s1_ref[...] = jnp.zeros_like(s1_ref)
        s2_ref[...] = jnp.zeros_like(s2_ref)

    s1_ref[...] += jnp.sum(h, axis=0, keepdims=True)
    s2_ref[...] += jnp.sum(h * h, axis=0, keepdims=True)


def _bn_relu_rows(h, s1, s2, n_all, g, b, n_valid):
    mean = s1 / n_all
    var = s2 / n_all - mean * mean
    inv = jax.lax.rsqrt(var + 1e-5)
    hn = jnp.maximum((h - mean) * inv * g + b, 0.0)
    rm = (_iota((h.shape[0], 1), 0) < n_valid).astype(_f32)
    return hn * rm, rm


def _rank_row(scolm, n):
    eye = (_iota((n, n), 0) == _iota((n, n), 1)).astype(_f32)
    srowm = _dgen(scolm, eye, ((0,), (0,)))
    gt = (scolm > srowm).astype(_f32)
    eq = (scolm == srowm).astype(_f32)
    lt = (_iota((n, n), 0) < _iota((n, n), 1)).astype(_f32)
    return jnp.sum(gt + eq * lt, axis=0, keepdims=True)


def _pool1_kernel(h_ref, s1_ref, s2_ref, g_ref, b_ref, pw_ref, a_ref,
                  hp_ref, a2_ref, x1_ref):
    hn, rm = _bn_relu_rows(h_ref[0], s1_ref[...], s2_ref[...], N_ALL1,
                           g_ref[...], b_ref[...], NPG)
    pw = pw_ref[...]
    nrm = jnp.sqrt(jnp.sum(pw * pw))
    scol = jnp.tanh(_dotd(hn, pw) / nrm)
    scolm = jnp.where(rm > 0, scol, -1e30)
    rankrow = _rank_row(scolm, NPP)
    riota = _iota((K1P, NPP), 0).astype(_f32)
    p = jnp.where((riota == rankrow) & (rankrow < float(K1)), 1.0, 0.0)
    hs = hn * scol
    hp = _dote(p, hs)
    hp_ref[0] = hp
    x1_ref[0] = jnp.sum(hp, axis=0, keepdims=True) / float(K1)
    bmat = _dgend(a_ref[0], p, ((1,), (1,)))
    a2_ref[0] = _dotd(p, bmat)


def _pool2_kernel(h_ref, s1_ref, s2_ref, g_ref, b_ref, pw_ref, x2_ref):
    hn, rm = _bn_relu_rows(h_ref[0], s1_ref[...], s2_ref[...], N_ALL2,
                           g_ref[...], b_ref[...], K1)
    pw = pw_ref[...]
    nrm = jnp.sqrt(jnp.sum(pw * pw))
    scol = jnp.tanh(_dotd(hn, pw) / nrm)
    scolm = jnp.where(rm > 0, scol, -1e30)
    rankrow = _rank_row(scolm, K1P)
    kept = jnp.where(rankrow < float(K2), 1.0, 0.0)
    hs = hn * scol
    x2_ref[0] = _dot(kept, hs) / float(K2)


def _pairmax(z, n_in):
    i0 = _iota((n_in, n_in // 2), 0)
    i1 = _iota((n_in, n_in // 2), 1)
    se = jnp.where(i0 == 2 * i1, 1.0, 0.0)
    so = jnp.where(i0 == 2 * i1 + 1, 1.0, 0.0)
    return jnp.maximum(_dot(z, se), _dot(z, so))


def _bn_batch_relu(m, g, b):
    mean = jnp.sum(m, axis=0, keepdims=True) / float(G)
    var = jnp.sum(m * m, axis=0, keepdims=True) / float(G) - mean * mean
    return jnp.maximum((m - mean) * jax.lax.rsqrt(var + 1e-5) * g + b, 0.0)


def _zprep_kernel(x1_ref, x2_ref, g1_ref, b1_ref, g2_ref, b2_ref, z_ref):
    z = x1_ref[...] + x2_ref[...]
    m = _pairmax(z, H)
    m = _bn_batch_relu(m, g1_ref[...], b1_ref[...])
    m = _pairmax(m, 512)
    z_ref[...] = _bn_batch_relu(m, g2_ref[...], b2_ref[...])


_NL = G * 256


def _shift(x, d):
    if d == 0:
        return x
    pad = jnp.zeros((x.shape[0], d), _f32)
    return jnp.concatenate([x[:, d:], pad], axis=1)


def _bn2d_masked(a, mask, count, g, b):
    am = a * mask
    mean = jnp.sum(am, axis=1, keepdims=True) / count
    var = jnp.sum(am * am, axis=1, keepdims=True) / count - mean * mean
    return (a - mean) * jax.lax.rsqrt(var + 1e-5) * g + b


def _head_kernel(z_ref, cw1_ref, cb1_ref, bg1_ref, bb1_ref, w2_ref, cb2_ref,
                 bg2_ref, bb2_ref, cw3_ref, cb3_ref, cw4_ref, cb4_ref,
                 bg3_ref, bb3_ref, w5_ref, cb5_ref, bg4_ref, bb4_ref,
                 cw6_ref, cb6_ref, out_ref):
    zrow = z_ref[...]
    q = _iota((1, _NL), 1) % 256
    hh = q // 16
    ww = q % 16
    m_a = jnp.where((hh % 2 == 0) & (ww % 2 == 0) & (hh <= 12) & (ww <= 12),
                    1.0, 0.0)
    m_b = jnp.where((hh % 4 == 0) & (ww % 4 == 0) & (hh <= 8) & (ww <= 8),
                    1.0, 0.0)

    a1 = jnp.maximum(_dot(cw1_ref[...], zrow) + cb1_ref[...], 0.0)
    ones = jnp.ones((1, _NL), _f32)
    a1 = _bn2d_masked(a1, ones, float(_NL), bg1_ref[...], bb1_ref[...])
    acc = jnp.zeros((64, _NL), _f32)
    for di in range(3):
        for dj in range(3):
            t = di * 3 + dj
            wt = w2_ref[t * 64:(t + 1) * 64, :]
            acc += _dotd(wt, _shift(a1, di * 16 + dj))
    a2 = jnp.maximum(acc + cb2_ref[...], 0.0)
    a2 = _bn2d_masked(a2, m_a, float(G * 49), bg2_ref[...], bb2_ref[...])
    a3 = jnp.maximum(_dotd(cw3_ref[...], a2) + cb3_ref[...], 0.0)
    mp = zrow
    first = True
    for di in range(3):
        for dj in range(3):
            s = _shift(zrow, di * 16 + dj)
            mp = s if first else jnp.maximum(mp, s)
            first = False
    z2 = (a3 + mp) * m_a

    a4 = jnp.maximum(_dot(cw4_ref[...], z2) + cb4_ref[...], 0.0)
    a4 = _bn2d_masked(a4, m_a, float(G * 49), bg3_ref[...], bb3_ref[...])
    acc5 = jnp.zeros((64, _NL), _f32)
    for di in range(3):
        for dj in range(3):
            t = di * 3 + dj
            wt = w5_ref[t * 64:(t + 1) * 64, :]
            acc5 += _dotd(wt, _shift(a4, 32 * di + 2 * dj))
    a5 = jnp.maximum(acc5 + cb5_ref[...], 0.0)
    a5 = _bn2d_masked(a5, m_b, float(G * 9), bg4_ref[...], bb4_ref[...])
    a6 = jnp.maximum(_dotd(cw6_ref[...], a5) + cb6_ref[...], 0.0)
    mp2 = z2
    first = True
    for di in range(3):
        for dj in range(3):
            s = _shift(z2, 32 * di + 2 * dj)
            mp2 = s if first else jnp.maximum(mp2, s)
            first = False
    out_ref[...] = (a6 + mp2) * m_b


def _fc_kernel(z_ref, fw_ref, fb_ref, f1w_ref, f1b_ref, out_ref):
    zm = z_ref[...]
    p_i = _iota((256, 16), 0)
    f_i = _iota((256, 16), 1)
    tgt = 64 * (f_i // 3) + 4 * (f_i % 3)
    sel = jnp.where((p_i == tgt) & (f_i < 9), 1.0, 0.0)
    z9 = _dot(zm, sel)
    hfc = jnp.maximum(_dotd(z9, fw_ref[...]) + fb_ref[...], 0.0)
    out_ref[...] = _dotd(hfc, f1w_ref[...]) + f1b_ref[...]


def kernel(x, edge_index, batch, Wrel1, brel1, Wroot1, g1, b1, p1w, Wrel2,
           brel2, Wroot2, g2, b2, p2w, G1, B1, G2, B2, cw1, cb1, bg1, bb1,
           cw2, cb2, bg2, bb2, cw3, cb3, cw4, cb4, bg3, bb3, cw5, cb5, bg4,
           bb4, cw6, cb6, fcW, fcb, fc1W, fc1b):
    f32 = _f32
    xp = jnp.pad(x.reshape(G, NPG, D), ((0, 0), (0, NPP - NPG), (0, 0)))

    adj = _adj_sc(edge_index[0].reshape(G, EPG),
                  edge_index[1].reshape(G, EPG)).reshape(G, NPP, NPP)

    def gconv(a, h_in, n, wr, wo, b, n_valid):
        return pl.pallas_call(
            functools.partial(_gconv_kernel, n_valid),
            grid=(G,),
            in_specs=[
                pl.BlockSpec((1, n, n), lambda g: (g, 0, 0)),
                pl.BlockSpec((1, n, h_in.shape[-1]), lambda g: (g, 0, 0)),
                pl.BlockSpec(wr.shape, lambda g: (0, 0)),
                pl.BlockSpec(wo.shape, lambda g: (0, 0)),
                pl.BlockSpec((1, H), lambda g: (0, 0)),
            ],
            out_specs=[
                pl.BlockSpec((1, n, H), lambda g: (g, 0, 0)),
                pl.BlockSpec((1, H), lambda g: (0, 0)),
                pl.BlockSpec((1, H), lambda g: (0, 0)),
            ],
            out_shape=[
                jax.ShapeDtypeStruct((G, n, H), f32),
                jax.ShapeDtypeStruct((1, H), f32),
                jax.ShapeDtypeStruct((1, H), f32),
            ],
        )(a, h_in, wr, wo, b.reshape(1, H))

    h1, s1a, s2a = gconv(adj, xp, NPP, Wrel1, Wroot1, brel1, NPG)

    hp, adj2, x1 = pl.pallas_call(
        _pool1_kernel,
        grid=(G,),
        in_specs=[
            pl.BlockSpec((1, NPP, H), lambda g: (g, 0, 0)),
            pl.BlockSpec((1, H), lambda g: (0, 0)),
            pl.BlockSpec((1, H), lambda g: (0, 0)),
            pl.BlockSpec((1, H), lambda g: (0, 0)),
            pl.BlockSpec((1, H), lambda g: (0, 0)),
            pl.BlockSpec((H, 1), lambda g: (0, 0)),
            pl.BlockSpec((1, NPP, NPP), lambda g: (g, 0, 0)),
        ],
        out_specs=[
            pl.BlockSpec((1, K1P, H), lambda g: (g, 0, 0)),
            pl.BlockSpec((1, K1P, K1P), lambda g: (g, 0, 0)),
            pl.BlockSpec((1, 1, H), lambda g: (g, 0, 0)),
        ],
        out_shape=[
            jax.ShapeDtypeStruct((G, K1P, H), f32),
            jax.ShapeDtypeStruct((G, K1P, K1P), f32),
            jax.ShapeDtypeStruct((G, 1, H), f32),
        ],
    )(h1, s1a, s2a, g1.reshape(1, H), b1.reshape(1, H), p1w.reshape(H, 1), adj)

    h2, s1b, s2b = gconv(adj2, hp, K1P, Wrel2, Wroot2, brel2, K1)

    x2 = pl.pallas_call(
        _pool2_kernel,
        grid=(G,),
        in_specs=[
            pl.BlockSpec((1, K1P, H), lambda g: (g, 0, 0)),
            pl.BlockSpec((1, H), lambda g: (0, 0)),
            pl.BlockSpec((1, H), lambda g: (0, 0)),
            pl.BlockSpec((1, H), lambda g: (0, 0)),
            pl.BlockSpec((1, H), lambda g: (0, 0)),
            pl.BlockSpec((H, 1), lambda g: (0, 0)),
        ],
        out_specs=pl.BlockSpec((1, 1, H), lambda g: (g, 0, 0)),
        out_shape=jax.ShapeDtypeStruct((G, 1, H), f32),
    )(h2, s1b, s2b, g2.reshape(1, H), b2.reshape(1, H), p2w.reshape(H, 1))

    z256 = pl.pallas_call(
        _zprep_kernel,
        in_specs=[pl.BlockSpec(s, lambda: tuple([0] * len(s)))
                  for s in [(G, H), (G, H), (1, 512), (1, 512),
                            (1, 256), (1, 256)]],
        out_specs=pl.BlockSpec((G, 256), lambda: (0, 0)),
        out_shape=jax.ShapeDtypeStruct((G, 256), f32),
    )(x1.reshape(G, H), x2.reshape(G, H), G1.reshape(1, 512), B1.reshape(1, 512),
      G2.reshape(1, 256), B2.reshape(1, 256))

    zrow = z256.reshape(1, _NL)
    w2taps = cw2.transpose(2, 3, 0, 1).reshape(9 * 64, 128)
    w5taps = cw5.transpose(2, 3, 0, 1).reshape(9 * 64, 128)
    head_ins = [
        zrow, cw1.reshape(128, 1), cb1.reshape(128, 1),
        bg1.reshape(128, 1), bb1.reshape(128, 1),
        w2taps, cb2.reshape(64, 1), bg2.reshape(64, 1), bb2.reshape(64, 1),
        cw3.reshape(1, 64), cb3.reshape(1, 1),
        cw4.reshape(128, 1), cb4.reshape(128, 1),
        bg3.reshape(128, 1), bb3.reshape(128, 1),
        w5taps, cb5.reshape(64, 1), bg4.reshape(64, 1), bb4.reshape(64, 1),
        cw6.reshape(1, 64), cb6.reshape(1, 1),
    ]
    z2row = pl.pallas_call(
        _head_kernel,
        in_specs=[pl.BlockSpec(a.shape, lambda: tuple([0] * a.ndim))
                  for a in head_ins],
        out_specs=pl.BlockSpec((1, _NL), lambda: (0, 0)),
        out_shape=jax.ShapeDtypeStruct((1, _NL), f32),
    )(*head_ins)

    fcWp = jnp.pad(fcW, ((0, 7), (0, 0)))
    f1wp = jnp.pad(fc1W, ((0, 0), (0, 118)))
    f1bp = jnp.pad(fc1b.reshape(1, 10), ((0, 0), (0, 118)))
    outp = pl.pallas_call(
        _fc_kernel,
        in_specs=[
            pl.BlockSpec((G, 256), lambda: (0, 0)),
            pl.BlockSpec((16, H), lambda: (0, 0)),
            pl.BlockSpec((1, H), lambda: (0, 0)),
            pl.BlockSpec((H, 128), lambda: (0, 0)),
            pl.BlockSpec((1, 128), lambda: (0, 0)),
        ],
        out_specs=pl.BlockSpec((G, 128), lambda: (0, 0)),
        out_shape=jax.ShapeDtypeStruct((G, 128), f32),
    )(z2row.reshape(G, 256), fcWp, fcb.reshape(1, H), f1wp, f1bp)
    return outp[:, :10]
